# baseline (speedup 1.0000x reference)
import functools

import jax
import jax.numpy as jnp
from jax.experimental import pallas as pl
from jax.experimental.pallas import tpu as pltpu

_BM = 256
_NS = 4


def _ln_relu(x, g, b, eps=1e-5):
    mu = jnp.mean(x, axis=-1, keepdims=True)
    var = jnp.mean((x - mu) ** 2, axis=-1, keepdims=True)
    y = (x - mu) * jax.lax.rsqrt(var + eps) * g + b
    return jnp.maximum(y, 0.0)


def _fused_kernel(*refs, nb, n):
    v_refs = refs[:_NS]
    a_refs = refs[_NS:2 * _NS]
    (eps_ref, w1t_ref, b1_ref, g1_ref, be1_ref,
     w2t_ref, b2_ref, g2_ref, be2_ref, o_ref, u_ref) = refs[2 * _NS:]
    i = pl.program_id(0)
    bf = jnp.bfloat16
    sw = n // _NS

    @pl.when(i < nb)
    def _phase_mm():
        acc = jnp.zeros((_BM, w1t_ref.shape[1]), jnp.float32)
        for s in range(_NS):
            acc += jnp.dot(v_refs[s][...].astype(bf),
                           w1t_ref[s * sw:(s + 1) * sw, :].astype(bf),
                           preferred_element_type=jnp.float32)
        u_ref[pl.ds(i * _BM, _BM), :] = acc.astype(bf)

    @pl.when(i >= nb)
    def _phase_gin():
        j = i - nb
        h = jnp.zeros((_BM, u_ref.shape[1]), jnp.float32)
        for s in range(_NS):
            h += jnp.dot(a_refs[s][...].astype(bf),
                         u_ref[s * sw:(s + 1) * sw, :],
                         preferred_element_type=jnp.float32)
        ublk = u_ref[pl.ds(j * _BM, _BM), :].astype(jnp.float32)
        h = h + eps_ref[0, 0] * ublk + b1_ref[...]
        h = _ln_relu(h, g1_ref[...], be1_ref[...])
        h2 = jnp.dot(h.astype(bf), w2t_ref[...].astype(bf),
                     preferred_element_type=jnp.float32)
        h2 = h2 + b2_ref[...]
        o_ref[...] = _ln_relu(h2, g2_ref[...], be2_ref[...])


def kernel(v, a, epsilon, W1, b1, g1, be1, W2, b2, g2, be2):
    n, _ = a.shape
    hid = W1.shape[0]
    out_dim = W2.shape[0]
    nb = n // _BM
    sw = n // _NS

    row = lambda x: x.reshape(1, -1)
    const = lambda i: (0, 0)
    v_specs = [pl.BlockSpec((_BM, sw), functools.partial(
        lambda i, s: (jnp.minimum(i, nb - 1), s), s=s)) for s in range(_NS)]
    a_specs = [pl.BlockSpec((_BM, sw), functools.partial(
        lambda i, s: (jnp.maximum(i - nb, 0), s), s=s)) for s in range(_NS)]
    out = pl.pallas_call(
        functools.partial(_fused_kernel, nb=nb, n=n),
        grid=(2 * nb,),
        in_specs=v_specs + a_specs + [
            pl.BlockSpec((1, 1), const),
            pl.BlockSpec((n, hid), const),
            pl.BlockSpec((1, hid), const),
            pl.BlockSpec((1, hid), const),
            pl.BlockSpec((1, hid), const),
            pl.BlockSpec((hid, out_dim), const),
            pl.BlockSpec((1, out_dim), const),
            pl.BlockSpec((1, out_dim), const),
            pl.BlockSpec((1, out_dim), const),
        ],
        out_specs=pl.BlockSpec((_BM, out_dim), lambda i: (jnp.maximum(i - nb, 0), 0)),
        out_shape=jax.ShapeDtypeStruct((n, out_dim), jnp.float32),
        scratch_shapes=[pltpu.VMEM((n, hid), jnp.bfloat16)],
    )(*([v] * _NS), *([a] * _NS), epsilon, W1.T, row(b1), row(g1), row(be1),
      W2.T, row(b2), row(g2), row(be2))
    return out


# speedup vs baseline: 1.1075x; 1.1075x over previous
import functools

import jax
import jax.numpy as jnp
from jax.experimental import pallas as pl
from jax.experimental.pallas import tpu as pltpu

_BK = 256


def _ln_relu(x, g, b, eps=1e-5):
    mu = jnp.mean(x, axis=-1, keepdims=True)
    var = jnp.mean((x - mu) ** 2, axis=-1, keepdims=True)
    y = (x - mu) * jax.lax.rsqrt(var + eps) * g + b
    return jnp.maximum(y, 0.0)


def _gin_kernel(v_ref, ac_ref, eps_ref, w1t_ref, b1_ref, g1_ref, be1_ref,
                w2t_ref, b2_ref, g2_ref, be2_ref, o_ref, u_ref, h_ref, *, nb):
    k = pl.program_id(0)
    bf = jnp.bfloat16

    u_k = jnp.dot(v_ref[...].astype(bf), w1t_ref[...],
                  preferred_element_type=jnp.float32)
    u_bf = u_k.astype(bf)
    u_ref[pl.ds(k * _BK, _BK), :] = u_bf
    part = jnp.dot(ac_ref[...].astype(bf), u_bf,
                   preferred_element_type=jnp.float32)

    @pl.when(k == 0)
    def _init():
        h_ref[...] = part

    @pl.when(k > 0)
    def _acc():
        h_ref[...] += part

    @pl.when(k == nb - 1)
    def _epilogue():
        h = h_ref[...] + eps_ref[0, 0] * u_ref[...].astype(jnp.float32)
        h = h + b1_ref[...]
        h = _ln_relu(h, g1_ref[...], be1_ref[...])
        h2 = jnp.dot(h.astype(bf), w2t_ref[...],
                     preferred_element_type=jnp.float32)
        h2 = h2 + b2_ref[...]
        o_ref[...] = _ln_relu(h2, g2_ref[...], be2_ref[...])


def kernel(v, a, epsilon, W1, b1, g1, be1, W2, b2, g2, be2):
    n, _ = a.shape
    hid = W1.shape[0]
    out_dim = W2.shape[0]
    nb = n // _BK

    row = lambda x: x.reshape(1, -1)
    const = lambda k: (0, 0)
    out = pl.pallas_call(
        functools.partial(_gin_kernel, nb=nb),
        grid=(nb,),
        in_specs=[
            pl.BlockSpec((_BK, n), lambda k: (k, 0)),
            pl.BlockSpec((n, _BK), lambda k: (0, k)),
            pl.BlockSpec((1, 1), const),
            pl.BlockSpec((n, hid), const),
            pl.BlockSpec((1, hid), const),
            pl.BlockSpec((1, hid), const),
            pl.BlockSpec((1, hid), const),
            pl.BlockSpec((hid, out_dim), const),
            pl.BlockSpec((1, out_dim), const),
            pl.BlockSpec((1, out_dim), const),
            pl.BlockSpec((1, out_dim), const),
        ],
        out_specs=pl.BlockSpec((n, out_dim), const),
        out_shape=jax.ShapeDtypeStruct((n, out_dim), jnp.float32),
        scratch_shapes=[pltpu.VMEM((n, hid), jnp.bfloat16),
                        pltpu.VMEM((n, hid), jnp.float32)],
    )(v, a, epsilon, W1.T.astype(jnp.bfloat16), row(b1), row(g1), row(be1),
      W2.T.astype(jnp.bfloat16), row(b2), row(g2), row(be2))
    return out
